# Initial kernel scaffold; baseline (speedup 1.0000x reference)
import jax
import jax.numpy as jnp
from jax.experimental import pallas as pl


def kernel(node_emb, W1, root1, b1, W2, root2, b2, rel_emb, edge_index, edge_type, head_indices, tail_indices, relation_types):
    raise NotImplementedError("write your pallas kernel here")


# speedup vs baseline: 3.6343x; 3.6343x over previous
import functools

import jax
import jax.numpy as jnp
from jax.experimental import pallas as pl


def _pick_block(n, cap=2048):
    best = 8
    for b in range(8, cap + 1, 8):
        if n % b == 0:
            best = b
    return best


def _layer_body(nr, din, relu, x_ref, a_ref, inv_ref, root_ref, w_ref, b_ref,
                o_ref):
    acc = jnp.dot(x_ref[...], root_ref[...],
                  preferred_element_type=jnp.float32) + b_ref[...]
    for r in range(nr):
        ar = a_ref[:, r * din:(r + 1) * din] * inv_ref[:, r:r + 1]
        acc = acc + jnp.dot(ar, w_ref[r], preferred_element_type=jnp.float32)
    if relu:
        acc = jnp.maximum(acc, 0.0)
    o_ref[...] = acc


def _fused_layer(x, a, inv, root, w, b, relu):
    n, din = x.shape
    nr, _, dout = w.shape
    blk = _pick_block(n)
    grid = (n // blk,)
    body = functools.partial(_layer_body, nr, din, relu)
    return pl.pallas_call(
        body,
        grid=grid,
        in_specs=[
            pl.BlockSpec((blk, din), lambda i: (i, 0)),
            pl.BlockSpec((blk, nr * din), lambda i: (i, 0)),
            pl.BlockSpec((blk, nr), lambda i: (i, 0)),
            pl.BlockSpec((din, dout), lambda i: (0, 0)),
            pl.BlockSpec((nr, din, dout), lambda i: (0, 0, 0)),
            pl.BlockSpec((1, dout), lambda i: (0, 0)),
        ],
        out_specs=pl.BlockSpec((blk, dout), lambda i: (i, 0)),
        out_shape=jax.ShapeDtypeStruct((n, dout), jnp.float32),
    )(x, a, inv, root, w, b)


def _distmult_body(h_ref, r_ref, t_ref, o_ref):
    o_ref[...] = jnp.sum(h_ref[...] * r_ref[...] * t_ref[...], axis=1)


def _distmult(head, rel, tail):
    b, d = head.shape
    blk = _pick_block(b)
    return pl.pallas_call(
        _distmult_body,
        grid=(b // blk,),
        in_specs=[
            pl.BlockSpec((blk, d), lambda i: (i, 0)),
            pl.BlockSpec((blk, d), lambda i: (i, 0)),
            pl.BlockSpec((blk, d), lambda i: (i, 0)),
        ],
        out_specs=pl.BlockSpec((blk,), lambda i: (i,)),
        out_shape=jax.ShapeDtypeStruct((b,), jnp.float32),
    )(head, rel, tail)


def kernel(node_emb, W1, root1, b1, W2, root2, b2, rel_emb, edge_index,
           edge_type, head_indices, tail_indices, relation_types):
    n, din = node_emb.shape
    nr = W1.shape[0]
    src = edge_index[0].astype(jnp.int32)
    dst = edge_index[1].astype(jnp.int32)
    et = edge_type.astype(jnp.int32)
    seg = dst * nr + et
    nseg = n * nr

    ones = jnp.ones(src.shape, jnp.float32)
    cnt = jax.ops.segment_sum(ones, seg, num_segments=nseg)
    inv = (1.0 / jnp.maximum(cnt, 1.0)).reshape(n, nr)

    a1 = jax.ops.segment_sum(jnp.take(node_emb, src, axis=0), seg,
                             num_segments=nseg).reshape(n, nr * din)
    h = _fused_layer(node_emb, a1, inv, root1, W1, b1.reshape(1, -1), True)

    dhid = h.shape[1]
    a2 = jax.ops.segment_sum(jnp.take(h, src, axis=0), seg,
                             num_segments=nseg).reshape(n, nr * dhid)
    h2 = _fused_layer(h, a2, inv, root2, W2, b2.reshape(1, -1), False)

    head = jnp.take(h2, head_indices.astype(jnp.int32), axis=0)
    tail = jnp.take(h2, tail_indices.astype(jnp.int32), axis=0)
    rel = jnp.take(rel_emb, relation_types.astype(jnp.int32), axis=0)
    return _distmult(head, rel, tail)


# speedup vs baseline: 4.1177x; 1.1330x over previous
import functools

import jax
import jax.numpy as jnp
from jax.experimental import pallas as pl


def _pick_block(n, cap=2048):
    best = 8
    for b in range(8, cap + 1, 8):
        if n % b == 0:
            best = b
    return best


def _layer_body(nr, din, relu, x_ref, a_ref, inv_ref, root_ref, w_ref, b_ref,
                o_ref):
    acc = jnp.dot(x_ref[...], root_ref[...],
                  preferred_element_type=jnp.float32) + b_ref[...]
    for r in range(nr):
        ar = a_ref[:, r * din:(r + 1) * din] * inv_ref[:, r:r + 1]
        acc = acc + jnp.dot(ar, w_ref[r], preferred_element_type=jnp.float32)
    if relu:
        acc = jnp.maximum(acc, 0.0)
    o_ref[...] = acc


def _fused_layer(x, a, inv, root, w, b, relu):
    n, din = x.shape
    nr, _, dout = w.shape
    blk = _pick_block(n)
    grid = (n // blk,)
    body = functools.partial(_layer_body, nr, din, relu)
    return pl.pallas_call(
        body,
        grid=grid,
        in_specs=[
            pl.BlockSpec((blk, din), lambda i: (i, 0)),
            pl.BlockSpec((blk, nr * din), lambda i: (i, 0)),
            pl.BlockSpec((blk, nr), lambda i: (i, 0)),
            pl.BlockSpec((din, dout), lambda i: (0, 0)),
            pl.BlockSpec((nr, din, dout), lambda i: (0, 0, 0)),
            pl.BlockSpec((1, dout), lambda i: (0, 0)),
        ],
        out_specs=pl.BlockSpec((blk, dout), lambda i: (i, 0)),
        out_shape=jax.ShapeDtypeStruct((n, dout), jnp.float32),
    )(x, a, inv, root, w, b)


def _distmult_body(h_ref, r_ref, t_ref, o_ref):
    o_ref[...] = jnp.sum(h_ref[...] * r_ref[...] * t_ref[...], axis=1)


def _distmult(head, rel, tail):
    b, d = head.shape
    blk = _pick_block(b)
    return pl.pallas_call(
        _distmult_body,
        grid=(b // blk,),
        in_specs=[
            pl.BlockSpec((blk, d), lambda i: (i, 0)),
            pl.BlockSpec((blk, d), lambda i: (i, 0)),
            pl.BlockSpec((blk, d), lambda i: (i, 0)),
        ],
        out_specs=pl.BlockSpec((blk,), lambda i: (i,)),
        out_shape=jax.ShapeDtypeStruct((b,), jnp.float32),
    )(head, rel, tail)


def kernel(node_emb, W1, root1, b1, W2, root2, b2, rel_emb, edge_index,
           edge_type, head_indices, tail_indices, relation_types):
    n, din = node_emb.shape
    nr = W1.shape[0]
    src = edge_index[0].astype(jnp.int32)
    dst = edge_index[1].astype(jnp.int32)
    et = edge_type.astype(jnp.int32)
    seg = dst * nr + et
    nseg = n * nr

    x_aug = jnp.concatenate(
        [node_emb, jnp.ones((n, 1), jnp.float32)], axis=1)
    g1 = jax.ops.segment_sum(jnp.take(x_aug, src, axis=0), seg,
                             num_segments=nseg)
    cnt = g1[:, din]
    inv = (1.0 / jnp.maximum(cnt, 1.0)).reshape(n, nr)
    a1 = g1[:, :din].reshape(n, nr * din)
    h = _fused_layer(node_emb, a1, inv, root1, W1, b1.reshape(1, -1), True)

    dhid = h.shape[1]
    a2 = jax.ops.segment_sum(jnp.take(h, src, axis=0), seg,
                             num_segments=nseg).reshape(n, nr * dhid)
    h2 = _fused_layer(h, a2, inv, root2, W2, b2.reshape(1, -1), False)

    head = jnp.take(h2, head_indices.astype(jnp.int32), axis=0)
    tail = jnp.take(h2, tail_indices.astype(jnp.int32), axis=0)
    rel = jnp.take(rel_emb, relation_types.astype(jnp.int32), axis=0)
    return _distmult(head, rel, tail)


# speedup vs baseline: 4.3545x; 1.0575x over previous
import functools

import jax
import jax.numpy as jnp
from jax.experimental import pallas as pl


def _pick_block(n, cap=2048):
    best = 8
    for b in range(8, cap + 1, 8):
        if n % b == 0:
            best = b
    return best


def _layer_body(nr, din, relu, x_ref, a_ref, inv_ref, root_ref, w_ref, b_ref,
                o_ref):
    acc = jnp.dot(x_ref[...], root_ref[...],
                  preferred_element_type=jnp.float32) + b_ref[...]
    for r in range(nr):
        ar = a_ref[:, r * din:(r + 1) * din] * inv_ref[:, r:r + 1]
        acc = acc + jnp.dot(ar, w_ref[r], preferred_element_type=jnp.float32)
    if relu:
        acc = jnp.maximum(acc, 0.0)
    o_ref[...] = acc


def _fused_layer(x, a, inv, root, w, b, relu):
    n, din = x.shape
    nr, _, dout = w.shape
    blk = _pick_block(n)
    grid = (n // blk,)
    body = functools.partial(_layer_body, nr, din, relu)
    return pl.pallas_call(
        body,
        grid=grid,
        in_specs=[
            pl.BlockSpec((blk, din), lambda i: (i, 0)),
            pl.BlockSpec((blk, nr * din), lambda i: (i, 0)),
            pl.BlockSpec((blk, nr), lambda i: (i, 0)),
            pl.BlockSpec((din, dout), lambda i: (0, 0)),
            pl.BlockSpec((nr, din, dout), lambda i: (0, 0, 0)),
            pl.BlockSpec((1, dout), lambda i: (0, 0)),
        ],
        out_specs=pl.BlockSpec((blk, dout), lambda i: (i, 0)),
        out_shape=jax.ShapeDtypeStruct((n, dout), jnp.float32),
    )(x, a, inv, root, w, b)


def _distmult_body(h_ref, r_ref, t_ref, o_ref):
    o_ref[...] = jnp.sum(h_ref[...] * r_ref[...] * t_ref[...], axis=1)


def _distmult(head, rel, tail):
    b, d = head.shape
    blk = _pick_block(b)
    return pl.pallas_call(
        _distmult_body,
        grid=(b // blk,),
        in_specs=[
            pl.BlockSpec((blk, d), lambda i: (i, 0)),
            pl.BlockSpec((blk, d), lambda i: (i, 0)),
            pl.BlockSpec((blk, d), lambda i: (i, 0)),
        ],
        out_specs=pl.BlockSpec((blk,), lambda i: (i,)),
        out_shape=jax.ShapeDtypeStruct((b,), jnp.float32),
    )(head, rel, tail)


def kernel(node_emb, W1, root1, b1, W2, root2, b2, rel_emb, edge_index,
           edge_type, head_indices, tail_indices, relation_types):
    n, din = node_emb.shape
    nr = W1.shape[0]
    src = edge_index[0].astype(jnp.int32)
    dst = edge_index[1].astype(jnp.int32)
    et = edge_type.astype(jnp.int32)
    seg = dst * nr + et
    nseg = n * nr

    x_aug = jnp.concatenate(
        [node_emb, jnp.ones((n, 1), jnp.float32)], axis=1)
    g1 = jax.ops.segment_sum(jnp.take(x_aug, src, axis=0, mode="clip"), seg,
                             num_segments=nseg)
    cnt = g1[:, din]
    inv = (1.0 / jnp.maximum(cnt, 1.0)).reshape(n, nr)
    a1 = g1[:, :din].reshape(n, nr * din)
    h = _fused_layer(node_emb, a1, inv, root1, W1, b1.reshape(1, -1), True)

    dhid = h.shape[1]
    a2 = jax.ops.segment_sum(jnp.take(h, src, axis=0, mode="clip"), seg,
                             num_segments=nseg).reshape(n, nr * dhid)
    h2 = _fused_layer(h, a2, inv, root2, W2, b2.reshape(1, -1), False)

    head = jnp.take(h2, head_indices.astype(jnp.int32), axis=0, mode="clip")
    tail = jnp.take(h2, tail_indices.astype(jnp.int32), axis=0, mode="clip")
    rel = jnp.take(rel_emb, relation_types.astype(jnp.int32), axis=0,
                   mode="clip")
    return _distmult(head, rel, tail)
